# Initial kernel scaffold; baseline (speedup 1.0000x reference)
import functools

import jax
import jax.numpy as jnp
from jax import lax
from jax.experimental import pallas as pl
from jax.experimental.pallas import tpu as pltpu
from jax.experimental.pallas import tpu_sc as plsc

B = 16384
NC, NS = 2, 16
NW = NC * NS
BPW = B // NW
NCHUNK = BPW // 128
IDX_ROWS = B // 128

_mesh = plsc.VectorSubcoreMesh(core_axis_name="c", subcore_axis_name="s")


@functools.partial(
    pl.kernel,
    mesh=_mesh,
    out_type=[
        jax.ShapeDtypeStruct((B, 16), jnp.float32),
        jax.ShapeDtypeStruct((B, 16), jnp.float32),
        jax.ShapeDtypeStruct((B, 16), jnp.float32),
    ],
    scratch_types=[
        pltpu.VMEM((NCHUNK, 128), jnp.int32),
        pltpu.VMEM((NCHUNK, 128), jnp.int32),
        pltpu.VMEM((NCHUNK, 128), jnp.int32),
        pltpu.VMEM((NCHUNK, 128), jnp.int32),
        pltpu.VMEM((NCHUNK, 128), jnp.int32),
        pltpu.VMEM((BPW, 16), jnp.float32),
        pltpu.VMEM((BPW, 16), jnp.float32),
        pltpu.VMEM((BPW, 16), jnp.float32),
        pltpu.SemaphoreType.DMA,
    ],
)
def _sc_gather(uid_hbm, iid_hbm, g_hbm, h_hbm, user_emb, item_emb, gh_table,
               out_u, out_i, out_gh,
               idx_u, idx_i, g_v, h_v, idx_gh, rows_u, rows_i, rows_gh, sem):
    wid = lax.axis_index("s") * NC + lax.axis_index("c")
    r0 = wid * NCHUNK
    pltpu.sync_copy(uid_hbm.at[pl.ds(r0, NCHUNK)], idx_u)
    pltpu.sync_copy(iid_hbm.at[pl.ds(r0, NCHUNK)], idx_i)
    pltpu.sync_copy(g_hbm.at[pl.ds(r0, NCHUNK)], g_v)
    pltpu.sync_copy(h_hbm.at[pl.ds(r0, NCHUNK)], h_v)
    for j in range(NCHUNK):
        for t in range(128 // 16):
            s = pl.ds(t * 16, 16)
            idx_gh[j, s] = g_v[j, s] * 24 + h_v[j, s]
    cps = []
    for j in range(NCHUNK):
        d = pl.ds(j * 128, 128)
        cps.append(pltpu.async_copy(user_emb.at[idx_u.at[j]], rows_u.at[d], sem))
        cps.append(pltpu.async_copy(item_emb.at[idx_i.at[j]], rows_i.at[d], sem))
        cps.append(pltpu.async_copy(gh_table.at[idx_gh.at[j]], rows_gh.at[d], sem))
    for cp in cps:
        cp.wait()
    base = wid * BPW
    pltpu.sync_copy(rows_u, out_u.at[pl.ds(base, BPW)])
    pltpu.sync_copy(rows_i, out_i.at[pl.ds(base, BPW)])
    pltpu.sync_copy(rows_gh, out_gh.at[pl.ds(base, BPW)])


BLK = 2048


def _mlp_body(u_ref, i_ref, gh_ref, w1_ref, b1_ref, w2_ref, b2_ref, out_ref):
    w1 = w1_ref[...]
    h = (jnp.dot(u_ref[...], w1[0:16, :], preferred_element_type=jnp.float32)
         + jnp.dot(i_ref[...], w1[16:32, :], preferred_element_type=jnp.float32)
         + jnp.dot(gh_ref[...], w1[32:48, :], preferred_element_type=jnp.float32)
         + b1_ref[...])
    h = jnp.maximum(h, 0.0)
    out_ref[...] = jnp.dot(h, w2_ref[...], preferred_element_type=jnp.float32) + b2_ref[...]


_mlp = pl.pallas_call(
    _mlp_body,
    grid=(B // BLK,),
    in_specs=[
        pl.BlockSpec((BLK, 16), lambda k: (k, 0)),
        pl.BlockSpec((BLK, 16), lambda k: (k, 0)),
        pl.BlockSpec((BLK, 16), lambda k: (k, 0)),
        pl.BlockSpec((48, 32), lambda k: (0, 0)),
        pl.BlockSpec((1, 32), lambda k: (0, 0)),
        pl.BlockSpec((32, 1), lambda k: (0, 0)),
        pl.BlockSpec((1, 1), lambda k: (0, 0)),
    ],
    out_specs=pl.BlockSpec((BLK, 1), lambda k: (k, 0)),
    out_shape=jax.ShapeDtypeStruct((B, 1), jnp.float32),
)


def kernel(user_id, item_id, gender, hour, user_emb, item_emb, gender_emb,
           hour_emb, W1, b1, W2, b2):
    uid2d = user_id.astype(jnp.int32).reshape(IDX_ROWS, 128)
    iid2d = item_id.astype(jnp.int32).reshape(IDX_ROWS, 128)
    g2d = gender.astype(jnp.int32).reshape(IDX_ROWS, 128)
    h2d = hour.astype(jnp.int32).reshape(IDX_ROWS, 128)
    ar = jnp.arange(48)
    gh_table = jnp.concatenate(
        [jnp.take(gender_emb, ar // 24, axis=0),
         jnp.take(hour_emb, ar % 24, axis=0),
         jnp.zeros((48, 8), jnp.float32)], axis=1)
    W1p = jnp.concatenate([W1, jnp.zeros((8, 32), W1.dtype)], axis=0)
    u, i, gh = _sc_gather(uid2d, iid2d, g2d, h2d, user_emb, item_emb, gh_table)
    return _mlp(u, i, gh, W1p, b1.reshape(1, 32), W2, b2.reshape(1, 1))


# speedup vs baseline: 2.9434x; 2.9434x over previous
import functools

import jax
import jax.numpy as jnp
from jax import lax
from jax.experimental import pallas as pl
from jax.experimental.pallas import tpu as pltpu
from jax.experimental.pallas import tpu_sc as plsc

B = 16384
NC, NS = 2, 16
NW = NC * NS
BPW = B // NW
NCHUNK = BPW // 128
IDX_ROWS = B // 128

_mesh = plsc.VectorSubcoreMesh(core_axis_name="c", subcore_axis_name="s")


@functools.partial(
    pl.kernel,
    mesh=_mesh,
    compiler_params=pltpu.CompilerParams(use_tc_tiling_on_sc=False),
    out_type=[
        jax.ShapeDtypeStruct((B, 16), jnp.float32),
        jax.ShapeDtypeStruct((B, 16), jnp.float32),
        jax.ShapeDtypeStruct((B, 16), jnp.float32),
    ],
    scratch_types=[
        pltpu.VMEM((NCHUNK, 128), jnp.int32),
        pltpu.VMEM((NCHUNK, 128), jnp.int32),
        pltpu.VMEM((NCHUNK, 128), jnp.int32),
        pltpu.VMEM((NCHUNK, 128), jnp.int32),
        pltpu.VMEM((NCHUNK, 128), jnp.int32),
        pltpu.VMEM((BPW, 16), jnp.float32),
        pltpu.VMEM((BPW, 16), jnp.float32),
        pltpu.VMEM((BPW, 16), jnp.float32),
        pltpu.SemaphoreType.DMA,
    ],
)
def _sc_gather(uid_hbm, iid_hbm, g_hbm, h_hbm, user_emb, item_emb, gh_table,
               out_u, out_i, out_gh,
               idx_u, idx_i, g_v, h_v, idx_gh, rows_u, rows_i, rows_gh, sem):
    wid = lax.axis_index("s") * NC + lax.axis_index("c")
    r0 = wid * NCHUNK
    pltpu.sync_copy(uid_hbm.at[pl.ds(r0, NCHUNK)], idx_u)
    pltpu.sync_copy(iid_hbm.at[pl.ds(r0, NCHUNK)], idx_i)
    pltpu.sync_copy(g_hbm.at[pl.ds(r0, NCHUNK)], g_v)
    pltpu.sync_copy(h_hbm.at[pl.ds(r0, NCHUNK)], h_v)
    for j in range(NCHUNK):
        for t in range(128 // 16):
            s = pl.ds(t * 16, 16)
            idx_gh[j, s] = g_v[j, s] * 24 + h_v[j, s]
    cps = []
    for j in range(NCHUNK):
        d = pl.ds(j * 128, 128)
        cps.append(pltpu.async_copy(user_emb.at[idx_u.at[j]], rows_u.at[d], sem))
        cps.append(pltpu.async_copy(item_emb.at[idx_i.at[j]], rows_i.at[d], sem))
        cps.append(pltpu.async_copy(gh_table.at[idx_gh.at[j]], rows_gh.at[d], sem))
    for cp in cps:
        cp.wait()
    base = wid * BPW
    pltpu.sync_copy(rows_u, out_u.at[pl.ds(base, BPW)])
    pltpu.sync_copy(rows_i, out_i.at[pl.ds(base, BPW)])
    pltpu.sync_copy(rows_gh, out_gh.at[pl.ds(base, BPW)])


BLK = 2048


def _mlp_body(u_ref, i_ref, gh_ref, w1_ref, b1_ref, w2_ref, b2_ref, out_ref):
    w1 = w1_ref[...]
    h = (jnp.dot(u_ref[...], w1[0:16, :], preferred_element_type=jnp.float32)
         + jnp.dot(i_ref[...], w1[16:32, :], preferred_element_type=jnp.float32)
         + jnp.dot(gh_ref[...], w1[32:48, :], preferred_element_type=jnp.float32)
         + b1_ref[...])
    h = jnp.maximum(h, 0.0)
    out_ref[...] = jnp.dot(h, w2_ref[...], preferred_element_type=jnp.float32) + b2_ref[...]


_mlp = pl.pallas_call(
    _mlp_body,
    grid=(B // BLK,),
    in_specs=[
        pl.BlockSpec((BLK, 16), lambda k: (k, 0)),
        pl.BlockSpec((BLK, 16), lambda k: (k, 0)),
        pl.BlockSpec((BLK, 16), lambda k: (k, 0)),
        pl.BlockSpec((48, 32), lambda k: (0, 0)),
        pl.BlockSpec((1, 32), lambda k: (0, 0)),
        pl.BlockSpec((32, 1), lambda k: (0, 0)),
        pl.BlockSpec((1, 1), lambda k: (0, 0)),
    ],
    out_specs=pl.BlockSpec((BLK, 1), lambda k: (k, 0)),
    out_shape=jax.ShapeDtypeStruct((B, 1), jnp.float32),
)


def kernel(user_id, item_id, gender, hour, user_emb, item_emb, gender_emb,
           hour_emb, W1, b1, W2, b2):
    uid2d = user_id.astype(jnp.int32).reshape(IDX_ROWS, 128)
    iid2d = item_id.astype(jnp.int32).reshape(IDX_ROWS, 128)
    g2d = gender.astype(jnp.int32).reshape(IDX_ROWS, 128)
    h2d = hour.astype(jnp.int32).reshape(IDX_ROWS, 128)
    ar = jnp.arange(48)
    gh_table = jnp.concatenate(
        [jnp.take(gender_emb, ar // 24, axis=0),
         jnp.take(hour_emb, ar % 24, axis=0),
         jnp.zeros((48, 8), jnp.float32)], axis=1)
    W1p = jnp.concatenate([W1, jnp.zeros((8, 32), W1.dtype)], axis=0)
    u, i, gh = _sc_gather(uid2d, iid2d, g2d, h2d, user_emb, item_emb, gh_table)
    return _mlp(u, i, gh, W1p, b1.reshape(1, 32), W2, b2.reshape(1, 1))


# speedup vs baseline: 4.7595x; 1.6170x over previous
import functools

import jax
import jax.numpy as jnp
from jax import lax
from jax.experimental import pallas as pl
from jax.experimental.pallas import tpu as pltpu
from jax.experimental.pallas import tpu_sc as plsc

B = 16384
NC, NS = 2, 16
NW = NC * NS
BPW = B // NW
NCHUNK = BPW // 128

_mesh = plsc.VectorSubcoreMesh(core_axis_name="c", subcore_axis_name="s")


@functools.partial(
    pl.kernel,
    mesh=_mesh,
    compiler_params=pltpu.CompilerParams(use_tc_tiling_on_sc=False),
    out_type=[
        jax.ShapeDtypeStruct((B, 16), jnp.float32),
        jax.ShapeDtypeStruct((B, 16), jnp.float32),
        jax.ShapeDtypeStruct((B, 16), jnp.float32),
    ],
    scratch_types=[
        pltpu.VMEM((NCHUNK, 128), jnp.int32),
        pltpu.VMEM((NCHUNK, 128), jnp.int32),
        pltpu.VMEM((NCHUNK, 128), jnp.int32),
        pltpu.VMEM((NCHUNK, 128), jnp.int32),
        pltpu.VMEM((NCHUNK, 128), jnp.int32),
        pltpu.VMEM((BPW, 16), jnp.float32),
        pltpu.VMEM((BPW, 16), jnp.float32),
        pltpu.VMEM((BPW, 16), jnp.float32),
        pltpu.SemaphoreType.DMA,
        pltpu.SemaphoreType.DMA,
        pltpu.SemaphoreType.DMA,
    ],
)
def _sc_gather(uid_hbm, iid_hbm, g_hbm, h_hbm, user_emb, item_emb, gh_table,
               out_u, out_i, out_gh,
               idx_u, idx_i, g_v, h_v, idx_gh, rows_u, rows_i, rows_gh,
               sem_idx, sem_gather, sem_store):
    wid = lax.axis_index("s") * NC + lax.axis_index("c")
    base = wid * BPW
    icps = []
    for j in range(NCHUNK):
        s = pl.ds(base + j * 128, 128)
        icps.append(pltpu.async_copy(uid_hbm.at[s], idx_u.at[j], sem_idx))
        icps.append(pltpu.async_copy(iid_hbm.at[s], idx_i.at[j], sem_idx))
        icps.append(pltpu.async_copy(g_hbm.at[s], g_v.at[j], sem_idx))
        icps.append(pltpu.async_copy(h_hbm.at[s], h_v.at[j], sem_idx))
    for cp in icps:
        cp.wait()
    for j in range(NCHUNK):
        for t in range(128 // 16):
            s = pl.ds(t * 16, 16)
            idx_gh[j, s] = g_v[j, s] * 24 + h_v[j, s]
    cps = []
    for j in range(NCHUNK):
        d = pl.ds(j * 128, 128)
        cps.append(pltpu.async_copy(user_emb.at[idx_u.at[j]], rows_u.at[d], sem_gather))
        cps.append(pltpu.async_copy(item_emb.at[idx_i.at[j]], rows_i.at[d], sem_gather))
        cps.append(pltpu.async_copy(gh_table.at[idx_gh.at[j]], rows_gh.at[d], sem_gather))
    for cp in cps:
        cp.wait()
    o = pl.ds(base, BPW)
    scps = [pltpu.async_copy(rows_u, out_u.at[o], sem_store),
            pltpu.async_copy(rows_i, out_i.at[o], sem_store),
            pltpu.async_copy(rows_gh, out_gh.at[o], sem_store)]
    for cp in scps:
        cp.wait()


ROWS = B // 8
RBLK = 256


def _mlp_body(u_ref, i_ref, gh_ref, w1u_ref, w1i_ref, w1gh_ref, b1_ref,
              w2_ref, b2_ref, out_ref):
    y = (jnp.dot(u_ref[...], w1u_ref[...], preferred_element_type=jnp.float32)
         + jnp.dot(i_ref[...], w1i_ref[...], preferred_element_type=jnp.float32)
         + jnp.dot(gh_ref[...], w1gh_ref[...], preferred_element_type=jnp.float32)
         + b1_ref[...])
    y = jnp.maximum(y, 0.0)
    o = jnp.dot(y, w2_ref[...], preferred_element_type=jnp.float32) + b2_ref[...]
    out_ref[...] = o


_mlp = pl.pallas_call(
    _mlp_body,
    grid=(ROWS // RBLK,),
    in_specs=[
        pl.BlockSpec((RBLK, 128), lambda k: (k, 0)),
        pl.BlockSpec((RBLK, 128), lambda k: (k, 0)),
        pl.BlockSpec((RBLK, 128), lambda k: (k, 0)),
        pl.BlockSpec((128, 256), lambda k: (0, 0)),
        pl.BlockSpec((128, 256), lambda k: (0, 0)),
        pl.BlockSpec((128, 256), lambda k: (0, 0)),
        pl.BlockSpec((1, 256), lambda k: (0, 0)),
        pl.BlockSpec((256, 8), lambda k: (0, 0)),
        pl.BlockSpec((1, 8), lambda k: (0, 0)),
    ],
    out_specs=pl.BlockSpec((RBLK, 8), lambda k: (k, 0)),
    out_shape=jax.ShapeDtypeStruct((ROWS, 8), jnp.float32),
)


def kernel(user_id, item_id, gender, hour, user_emb, item_emb, gender_emb,
           hour_emb, W1, b1, W2, b2):
    uid = user_id.astype(jnp.int32)
    iid = item_id.astype(jnp.int32)
    g = gender.astype(jnp.int32)
    h = hour.astype(jnp.int32)
    ar = jnp.arange(48)
    gh_table = jnp.concatenate(
        [jnp.take(gender_emb, ar // 24, axis=0),
         jnp.take(hour_emb, ar % 24, axis=0),
         jnp.zeros((48, 8), jnp.float32)], axis=1)
    u, i, gh = _sc_gather(uid, iid, g, h, user_emb, item_emb, gh_table)
    eye8 = jnp.eye(8, dtype=jnp.float32)
    w1gh = jnp.concatenate([W1[32:40], jnp.zeros((8, 32), W1.dtype)], axis=0)
    w1u_big = jnp.kron(eye8, W1[0:16])
    w1i_big = jnp.kron(eye8, W1[16:32])
    w1gh_big = jnp.kron(eye8, w1gh)
    b1_big = jnp.tile(b1, 8).reshape(1, 256)
    w2_big = jnp.kron(eye8, W2)
    b2_big = jnp.tile(b2, 8).reshape(1, 8)
    out = _mlp(u.reshape(ROWS, 128), i.reshape(ROWS, 128), gh.reshape(ROWS, 128),
               w1u_big, w1i_big, w1gh_big, b1_big, w2_big, b2_big)
    return out.reshape(B, 1)
